# baseline (speedup 1.0000x reference)
import functools

import jax
import jax.numpy as jnp
from jax import lax
from jax.experimental import pallas as pl
from jax.experimental.pallas import tpu as pltpu
from jax.experimental.pallas import tpu_sc as plsc

N = 320000
D = 128
S = 10000
SP = 10240
NC = 2
NS = 16
DC = D // NC
WC = DC + 16
CHUNK = 256
SUB = 128
UNITS = N // CHUNK
TRIPS = UNITS // NS
EXTRA = UNITS - TRIPS * NS
ZROWS = SP // NS


def _sc_segment_mean(rawret, labels2d):
    mesh = plsc.VectorSubcoreMesh(core_axis_name="c", subcore_axis_name="s")

    @functools.partial(
        pl.kernel,
        out_type=jax.ShapeDtypeStruct((SP, D), jnp.float32),
        mesh=mesh,
        compiler_params=pltpu.CompilerParams(use_tc_tiling_on_sc=False),
        scratch_types=[
            pltpu.VMEM((2, CHUNK, WC), jnp.float32),
            pltpu.VMEM((2, 2, SUB), jnp.int32),
            pltpu.VMEM((SUB, WC), jnp.float32),
            pltpu.VMEM_SHARED((SP, WC), jnp.float32),
            pltpu.SemaphoreType.DMA,
            pltpu.SemaphoreType.DMA,
        ],
    )
    def seg_mean(raw_hbm, lbl_hbm, out_hbm, rows_v, lbl_v, stage, acc_sh,
                 sem0, sem1):
        c = lax.axis_index("c")
        s = lax.axis_index("s")
        col0 = c * DC
        sems = (sem0, sem1)

        zero16 = jnp.zeros((16,), jnp.float32)
        one16 = jnp.ones((16,), jnp.float32)

        @pl.loop(0, CHUNK)
        def _(i):
            rows_v[0, i, pl.ds(DC, 16)] = one16
            rows_v[1, i, pl.ds(DC, 16)] = one16

        @pl.loop(0, SUB)
        def _(i):
            @pl.loop(0, WC // 16)
            def _(j):
                stage[i, pl.ds(j * 16, 16)] = zero16

        zrow = s * ZROWS
        for b in range(ZROWS // SUB):
            pltpu.sync_copy(stage, acc_sh.at[pl.ds(zrow + b * SUB, SUB)])
        plsc.subcore_barrier()

        my_first = TRIPS * s

        def rows_src(u):
            return raw_hbm.at[pl.ds(u * CHUNK, CHUNK), pl.ds(col0, DC)]

        def rows_dst(b):
            return rows_v.at[b].at[pl.ds(0, CHUNK), pl.ds(0, DC)]

        def lbl_src(u):
            return lbl_hbm.at[pl.ds(u * 2, 2)]

        def dma_in(u, b):
            pltpu.async_copy(rows_src(u), rows_dst(b), sems[b])
            pltpu.async_copy(lbl_src(u), lbl_v.at[b], sems[b])

        def dma_wait(u, b):
            pltpu.make_async_copy(rows_src(u), rows_dst(b), sems[b]).wait()
            pltpu.make_async_copy(lbl_src(u), lbl_v.at[b], sems[b]).wait()

        def scatter(b):
            for j in range(CHUNK // SUB):
                pltpu.sync_copy(rows_v.at[b].at[pl.ds(j * SUB, SUB)],
                                acc_sh.at[lbl_v.at[b].at[j]], add=True)

        dma_in(my_first, 0)

        @pl.loop(0, TRIPS // 2)
        def _(o):
            for b in range(2):
                t = 2 * o + b
                u = my_first + t

                dma_wait(u, b)

                @pl.when(t + 1 < TRIPS)
                def _():
                    dma_in(u + 1, 1 - b)

                scatter(b)

        @pl.when(s < EXTRA)
        def _():
            u = TRIPS * NS + s
            pltpu.sync_copy(rows_src(u), rows_dst(0))
            pltpu.sync_copy(lbl_src(u), lbl_v.at[0])
            scatter(0)

        plsc.subcore_barrier()

        for b in range(ZROWS // SUB):
            pltpu.sync_copy(acc_sh.at[pl.ds(zrow + b * SUB, SUB)], stage)

            @pl.loop(0, SUB)
            def _(i):
                cnt = stage[i, pl.ds(DC, 16)]
                r = one16 / jnp.maximum(cnt, one16)

                @pl.loop(0, DC // 16)
                def _(j):
                    stage[i, pl.ds(j * 16, 16)] = (
                        stage[i, pl.ds(j * 16, 16)] * r)

            pltpu.sync_copy(
                stage.at[pl.ds(0, SUB), pl.ds(0, DC)],
                out_hbm.at[pl.ds(zrow + b * SUB, SUB), pl.ds(col0, DC)])

    return seg_mean(rawret, labels2d)


def kernel(rawret, labels):
    labels2d = labels.reshape(N // 128, 128)
    means = _sc_segment_mean(rawret, labels2d)
    return means[:S]


# speedup vs baseline: 1.3279x; 1.3279x over previous
import functools

import jax
import jax.numpy as jnp
from jax import lax
from jax.experimental import pallas as pl
from jax.experimental.pallas import tpu as pltpu
from jax.experimental.pallas import tpu_sc as plsc

N = 320000
D = 128
S = 10000
SP = 10240
NC = 2
NS = 16
DC = D // NC
CHUNK = 256
SUB = 128
UNITS = N // CHUNK
TRIPS = UNITS // NS
EXTRA = UNITS - TRIPS * NS
ZROWS = SP // NS


def _sc_segment_mean(rawret, labels2d):
    mesh = plsc.VectorSubcoreMesh(core_axis_name="c", subcore_axis_name="s")

    @functools.partial(
        pl.kernel,
        out_type=jax.ShapeDtypeStruct((SP, D), jnp.float32),
        mesh=mesh,
        compiler_params=pltpu.CompilerParams(use_tc_tiling_on_sc=False,
                                             needs_layout_passes=False),
        scratch_types=[
            pltpu.VMEM((2, CHUNK, DC), jnp.float32),
            pltpu.VMEM((2, 2, SUB), jnp.int32),
            pltpu.VMEM((SUB, DC), jnp.float32),
            pltpu.VMEM((SP,), jnp.float32),
            pltpu.VMEM((NS, ZROWS), jnp.float32),
            pltpu.VMEM((ZROWS,), jnp.float32),
            pltpu.VMEM_SHARED((SP, DC), jnp.float32),
            pltpu.VMEM_SHARED((NS, SP), jnp.float32),
            pltpu.SemaphoreType.DMA,
            pltpu.SemaphoreType.DMA,
        ],
    )
    def seg_mean(raw_hbm, lbl_hbm, out_hbm, rows_v, lbl_v, stage, hist,
                 cslot, csum, acc_sh, cnt_sh, sem0, sem1):
        c = lax.axis_index("c")
        s = lax.axis_index("s")
        col0 = c * DC
        sems = (sem0, sem1)

        zero16 = jnp.zeros((16,), jnp.float32)
        one16 = jnp.ones((16,), jnp.float32)

        @pl.loop(0, SUB)
        def _(i):
            @pl.loop(0, DC // 16)
            def _(j):
                stage[i, pl.ds(j * 16, 16)] = zero16

        @pl.loop(0, SP // 16)
        def _(i):
            hist[pl.ds(i * 16, 16)] = zero16

        zrow = s * ZROWS
        for b in range(ZROWS // SUB):
            pltpu.sync_copy(stage, acc_sh.at[pl.ds(zrow + b * SUB, SUB)])
        plsc.subcore_barrier()

        my_first = TRIPS * s

        def rows_src(u):
            return raw_hbm.at[pl.ds(u * CHUNK, CHUNK), pl.ds(col0, DC)]

        def lbl_src(u):
            return lbl_hbm.at[pl.ds(u * 2, 2)]

        def dma_in(u, b):
            pltpu.async_copy(rows_src(u), rows_v.at[b], sems[b])
            pltpu.async_copy(lbl_src(u), lbl_v.at[b], sems[b])

        def dma_wait(u, b):
            pltpu.make_async_copy(rows_src(u), rows_v.at[b], sems[b]).wait()
            pltpu.make_async_copy(lbl_src(u), lbl_v.at[b], sems[b]).wait()

        def scatter(b):
            for j in range(CHUNK // SUB):
                for k in range(SUB // 16):
                    v = lbl_v[b, j, pl.ds(k * 16, 16)]
                    plsc.addupdate_scatter(hist, [v], one16)

            for j in range(CHUNK // SUB):
                pltpu.sync_copy(rows_v.at[b].at[pl.ds(j * SUB, SUB)],
                                acc_sh.at[lbl_v.at[b].at[j]], add=True)

        dma_in(my_first, 0)

        @pl.loop(0, TRIPS // 2)
        def _(o):
            for b in range(2):
                t = 2 * o + b
                u = my_first + t

                dma_wait(u, b)

                @pl.when(t + 1 < TRIPS)
                def _():
                    dma_in(u + 1, 1 - b)

                scatter(b)

        @pl.when(s < EXTRA)
        def _():
            u = TRIPS * NS + s
            pltpu.sync_copy(rows_src(u), rows_v.at[0])
            pltpu.sync_copy(lbl_src(u), lbl_v.at[0])
            scatter(0)

        pltpu.sync_copy(hist, cnt_sh.at[s])

        plsc.subcore_barrier()

        for t in range(NS):
            pltpu.sync_copy(cnt_sh.at[t].at[pl.ds(zrow, ZROWS)], cslot.at[t])

        @pl.loop(0, ZROWS // 16)
        def _(r):
            acc = cslot[0, pl.ds(r * 16, 16)]
            for t in range(1, NS):
                acc = acc + cslot[t, pl.ds(r * 16, 16)]
            csum[pl.ds(r * 16, 16)] = acc

        for b in range(ZROWS // SUB):
            pltpu.sync_copy(acc_sh.at[pl.ds(zrow + b * SUB, SUB)], stage)

            @pl.loop(0, SUB)
            def _(i):
                g = jnp.full((16,), b * SUB, jnp.int32) + i
                cnt = plsc.load_gather(csum, [g])
                r = one16 / jnp.maximum(cnt, one16)

                @pl.loop(0, DC // 16)
                def _(j):
                    stage[i, pl.ds(j * 16, 16)] = (
                        stage[i, pl.ds(j * 16, 16)] * r)

            pltpu.sync_copy(
                stage,
                out_hbm.at[pl.ds(zrow + b * SUB, SUB), pl.ds(col0, DC)])

    return seg_mean(rawret, labels2d)


def kernel(rawret, labels):
    labels2d = labels.reshape(N // 128, 128)
    means = _sc_segment_mean(rawret, labels2d)
    return means[:S]
